# baseline (speedup 1.0000x reference)
import functools

import jax
import jax.numpy as jnp
from jax import lax
from jax.experimental import pallas as pl
from jax.experimental.pallas import tpu as pltpu
from jax.experimental.pallas import tpu_sc as plsc

_N_SRC1 = 10000
_N_DST1 = 4000
_N_DST2 = 1000
_IN = 256
_HID = 128
_CLS = 40

_NC = 2
_NS = 16
_NW = _NC * _NS
_CH = 128

_E1P = 131072
_NCH1 = _E1P // (_NW * _CH)
_ND1P = 4096
_D1 = _HID

_E2P = 32768
_NCH2 = _E2P // (_NW * _CH)
_ND2P = 1024
_D2 = 48

_ZR = 64
_DW = 16


def _sc_degrees():
    mesh = plsc.VectorSubcoreMesh(core_axis_name="c", subcore_axis_name="s")
    rps1 = _ND1P // _NS
    rps2 = _ND2P // _NS

    def body(dst1_hbm, dst2_hbm, out1_hbm, out2_hbm, idx1, idx2, ones_v,
             zbuf, deg1_sh, deg2_sh, sem):
        cid = lax.axis_index("c")
        sid = lax.axis_index("s")
        wid = sid * _NC + cid

        @pl.loop(0, _ZR)
        def _(r):
            zbuf[r, pl.ds(0, _DW)] = jnp.zeros((_DW,), jnp.float32)

        @pl.loop(0, _CH)
        def _(r):
            ones_v[r, pl.ds(0, _DW)] = jnp.ones((_DW,), jnp.float32)

        @pl.loop(0, rps1, step=_ZR)
        def _(r):
            pltpu.sync_copy(zbuf, deg1_sh.at[pl.ds(sid * rps1 + r, _ZR)])

        pltpu.sync_copy(zbuf, deg2_sh.at[pl.ds(sid * rps2, _ZR)])
        pltpu.sync_copy(dst1_hbm.at[wid], idx1)
        pltpu.sync_copy(dst2_hbm.at[wid], idx2)
        plsc.subcore_barrier()

        @pl.loop(0, _NCH1)
        def _(c):
            pltpu.async_copy(ones_v, deg1_sh.at[idx1.at[c]], sem, add=True)

        @pl.loop(0, _NCH2)
        def _(c):
            pltpu.async_copy(ones_v, deg2_sh.at[idx2.at[c]], sem, add=True)

        @pl.loop(0, _NCH1 + _NCH2)
        def _(c):
            pltpu.make_async_copy(ones_v, deg1_sh.at[idx1.at[0]], sem).wait()

        plsc.subcore_barrier()
        pltpu.sync_copy(deg1_sh.at[pl.ds(sid * rps1, rps1)],
                        out1_hbm.at[cid, pl.ds(sid * rps1, rps1)])
        pltpu.sync_copy(deg2_sh.at[pl.ds(sid * rps2, rps2)],
                        out2_hbm.at[cid, pl.ds(sid * rps2, rps2)])

    return pl.kernel(
        body,
        out_type=(jax.ShapeDtypeStruct((_NC, _ND1P, _DW), jnp.float32),
                  jax.ShapeDtypeStruct((_NC, _ND2P, _DW), jnp.float32)),
        mesh=mesh,
        scratch_types=[
            pltpu.VMEM((_NCH1, _CH), jnp.int32),
            pltpu.VMEM((_NCH2, _CH), jnp.int32),
            pltpu.VMEM((_CH, _DW), jnp.float32),
            pltpu.VMEM((_ZR, _DW), jnp.float32),
            pltpu.VMEM_SHARED((_ND1P, _DW), jnp.float32),
            pltpu.VMEM_SHARED((_ND2P, _DW), jnp.float32),
            pltpu.SemaphoreType.DMA,
        ],
        compiler_params=pltpu.CompilerParams(use_tc_tiling_on_sc=False),
    )


def _sc_edge_agg(n_dst_pad, d, nch, tiled):
    rows_per_sub = n_dst_pad // _NS
    mesh = plsc.VectorSubcoreMesh(core_axis_name="c", subcore_axis_name="s")

    def body(m_hbm, src_hbm, dst_hbm, zeros_hbm, out_hbm, srcidx, dstidx,
             msgs0, msgs1, msgs2, msgs3, zbuf, agg_sh,
             gs0, gs1, gs2, gs3, ss0, ss1, ss2, ss3):
        msgs = [msgs0, msgs1, msgs2, msgs3]
        gs = [gs0, gs1, gs2, gs3]
        ss = [ss0, ss1, ss2, ss3]
        cid = lax.axis_index("c")
        sid = lax.axis_index("s")
        wid = sid * _NC + cid

        pltpu.sync_copy(zeros_hbm, zbuf)

        @pl.loop(0, rows_per_sub, step=_ZR)
        def _(r):
            pltpu.sync_copy(zbuf, agg_sh.at[pl.ds(sid * rows_per_sub + r, _ZR)])

        pltpu.sync_copy(src_hbm.at[wid], srcidx)
        pltpu.sync_copy(dst_hbm.at[wid], dstidx)
        plsc.subcore_barrier()

        for j in range(4):
            pltpu.async_copy(m_hbm.at[srcidx.at[j]], msgs[j], gs[j])

        @pl.loop(0, nch, step=4)
        def _(c):
            for j in range(4):
                jp = (j + 2) % 4
                pltpu.make_async_copy(m_hbm.at[srcidx.at[c + j]], msgs[j],
                                      gs[j]).wait()
                pltpu.async_copy(msgs[j], agg_sh.at[dstidx.at[c + j]], ss[j],
                                 add=True)

                @pl.when(c + j >= 2)
                def _():
                    pltpu.make_async_copy(
                        msgs[jp], agg_sh.at[dstidx.at[c + j - 2]],
                        ss[jp]).wait()

                @pl.when(jnp.logical_and(c + j >= 2, c + j + 2 < nch))
                def _():
                    pltpu.async_copy(m_hbm.at[srcidx.at[c + j + 2]], msgs[jp],
                                     gs[jp])

        pltpu.make_async_copy(msgs[2], agg_sh.at[dstidx.at[nch - 2]],
                              ss[2]).wait()
        pltpu.make_async_copy(msgs[3], agg_sh.at[dstidx.at[nch - 1]],
                              ss[3]).wait()
        plsc.subcore_barrier()
        pltpu.sync_copy(
            agg_sh.at[pl.ds(sid * rows_per_sub, rows_per_sub)],
            out_hbm.at[cid, pl.ds(sid * rows_per_sub, rows_per_sub)])

    return pl.kernel(
        body,
        out_type=jax.ShapeDtypeStruct((_NC, n_dst_pad, d), jnp.float32),
        mesh=mesh,
        scratch_types=[
            pltpu.VMEM((nch, _CH), jnp.int32),
            pltpu.VMEM((nch, _CH), jnp.int32),
            pltpu.VMEM((_CH, d), jnp.float32),
            pltpu.VMEM((_CH, d), jnp.float32),
            pltpu.VMEM((_CH, d), jnp.float32),
            pltpu.VMEM((_CH, d), jnp.float32),
            pltpu.VMEM((_ZR, d), jnp.float32),
            pltpu.VMEM_SHARED((n_dst_pad, d), jnp.float32),
            pltpu.SemaphoreType.DMA,
            pltpu.SemaphoreType.DMA,
            pltpu.SemaphoreType.DMA,
            pltpu.SemaphoreType.DMA,
            pltpu.SemaphoreType.DMA,
            pltpu.SemaphoreType.DMA,
            pltpu.SemaphoreType.DMA,
            pltpu.SemaphoreType.DMA,
        ],
        compiler_params=pltpu.CompilerParams(use_tc_tiling_on_sc=tiled),
    )


def _pad_edges(src, dst, e_pad, n_src, n_dst, n_dst_pad, nch):
    pad = e_pad - src.shape[0]
    i = jnp.arange(pad, dtype=jnp.int32)
    psrc = i % n_src
    pdst = n_dst + (i % (n_dst_pad - n_dst))
    src_p = jnp.concatenate([src, psrc]).reshape(_NW, nch, _CH)
    dst_p = jnp.concatenate([dst, pdst]).reshape(_NW, nch, _CH)
    return src_p, dst_p


def _tc_mm1(x, w1):
    def body(x_ref, w_ref, o_ref):
        o_ref[...] = jnp.dot(x_ref[...], w_ref[...],
                             preferred_element_type=jnp.float32,
                             precision=lax.Precision.DEFAULT)

    return pl.pallas_call(
        body,
        grid=(25,),
        in_specs=[pl.BlockSpec((400, _IN), lambda i: (i, 0)),
                  pl.BlockSpec((_IN, _HID), lambda i: (0, 0))],
        out_specs=pl.BlockSpec((400, _HID), lambda i: (i, 0)),
        out_shape=jax.ShapeDtypeStruct((_N_SRC1, _HID), jnp.float32),
    )(x, w1)


def _tc_mid(parts, degp, w2, b1):
    def body(p_ref, g_ref, w_ref, b_ref, o_ref):
        agg = p_ref[0, :_N_DST1] + p_ref[1, :_N_DST1]
        deg = g_ref[0, :_N_DST1, :1] + g_ref[1, :_N_DST1, :1]
        deg = jnp.maximum(deg, 1.0)
        h = agg / deg + b_ref[...]
        h = jnp.maximum(h, 0.0)
        m2 = jnp.dot(h, w_ref[...], preferred_element_type=jnp.float32,
                     precision=lax.Precision.DEFAULT)
        o_ref[:, :_CLS] = m2
        o_ref[:, _CLS:] = jnp.zeros((_N_DST1, _D2 - _CLS), jnp.float32)

    return pl.pallas_call(
        body,
        out_shape=jax.ShapeDtypeStruct((_N_DST1, _D2), jnp.float32),
    )(parts, degp, w2, b1.reshape(1, _HID))


def _tc_fin(parts, degp, b2):
    def body(p_ref, g_ref, b_ref, o_ref):
        agg = p_ref[0, :_N_DST2, :_CLS] + p_ref[1, :_N_DST2, :_CLS]
        deg = g_ref[0, :_N_DST2, :1] + g_ref[1, :_N_DST2, :1]
        deg = jnp.maximum(deg, 1.0)
        logits = agg / deg + b_ref[...]
        m = jnp.max(logits, axis=-1, keepdims=True)
        s = logits - m
        lse = jnp.log(jnp.sum(jnp.exp(s), axis=-1, keepdims=True))
        o_ref[...] = s - lse

    return pl.pallas_call(
        body,
        out_shape=jax.ShapeDtypeStruct((_N_DST2, _CLS), jnp.float32),
    )(parts, degp, b2.reshape(1, _CLS))


def kernel(x, edge_index_1, edge_index_2, W1, b1, W2, b2):
    s1, d1 = _pad_edges(edge_index_1[0], edge_index_1[1], _E1P,
                        _N_SRC1, _N_DST1, _ND1P, _NCH1)
    s2, d2 = _pad_edges(edge_index_2[0], edge_index_2[1], _E2P,
                        _N_DST1, _N_DST2, _ND2P, _NCH2)
    z1 = jnp.zeros((_ZR, _D1), jnp.float32)
    z2 = jnp.zeros((_ZR, _D2), jnp.float32)

    deg1p, deg2p = _sc_degrees()(d1, d2)
    m1 = _tc_mm1(x, W1)
    parts1 = _sc_edge_agg(_ND1P, _D1, _NCH1, True)(m1, s1, d1, z1)
    m2 = _tc_mid(parts1, deg1p, W2, b1)
    parts2 = _sc_edge_agg(_ND2P, _D2, _NCH2, False)(m2, s2, d2, z2)
    return _tc_fin(parts2, deg2p, b2)


# speedup vs baseline: 1.0690x; 1.0690x over previous
import functools

import jax
import jax.numpy as jnp
from jax import lax
from jax.experimental import pallas as pl
from jax.experimental.pallas import tpu as pltpu
from jax.experimental.pallas import tpu_sc as plsc

_N_SRC1 = 10000
_N_DST1 = 4000
_N_DST2 = 1000
_IN = 256
_HID = 128
_CLS = 40

_NC = 2
_NS = 16
_NW = _NC * _NS
_CH = 128

_E1P = 131072
_NCH1 = _E1P // (_NW * _CH)
_ND1P = 4096
_D1 = _HID

_E2P = 32768
_NCH2 = _E2P // (_NW * _CH)
_ND2P = 1024
_D2 = 48

_ZR = 64
_DW = 16


def _sc_degrees():
    mesh = plsc.VectorSubcoreMesh(core_axis_name="c", subcore_axis_name="s")
    rps1 = _ND1P // _NS
    rps2 = _ND2P // _NS

    def body(dst1_hbm, dst2_hbm, out1_hbm, out2_hbm, idx1, idx2, ones_v,
             zbuf, deg1_sh, deg2_sh, sem):
        cid = lax.axis_index("c")
        sid = lax.axis_index("s")
        wid = sid * _NC + cid

        @pl.loop(0, _ZR)
        def _(r):
            zbuf[r, pl.ds(0, _DW)] = jnp.zeros((_DW,), jnp.float32)

        @pl.loop(0, _CH)
        def _(r):
            ones_v[r, pl.ds(0, _DW)] = jnp.ones((_DW,), jnp.float32)

        @pl.loop(0, rps1, step=_ZR)
        def _(r):
            pltpu.sync_copy(zbuf, deg1_sh.at[pl.ds(sid * rps1 + r, _ZR)])

        pltpu.sync_copy(zbuf, deg2_sh.at[pl.ds(sid * rps2, _ZR)])
        pltpu.sync_copy(dst1_hbm.at[wid], idx1)
        pltpu.sync_copy(dst2_hbm.at[wid], idx2)
        plsc.subcore_barrier()

        @pl.loop(0, _NCH1)
        def _(c):
            pltpu.async_copy(ones_v, deg1_sh.at[idx1.at[c]], sem, add=True)

        @pl.loop(0, _NCH2)
        def _(c):
            pltpu.async_copy(ones_v, deg2_sh.at[idx2.at[c]], sem, add=True)

        @pl.loop(0, _NCH1 + _NCH2)
        def _(c):
            pltpu.make_async_copy(ones_v, deg1_sh.at[idx1.at[0]], sem).wait()

        plsc.subcore_barrier()
        pltpu.sync_copy(deg1_sh.at[pl.ds(sid * rps1, rps1)],
                        out1_hbm.at[cid, pl.ds(sid * rps1, rps1)])
        pltpu.sync_copy(deg2_sh.at[pl.ds(sid * rps2, rps2)],
                        out2_hbm.at[cid, pl.ds(sid * rps2, rps2)])

    return pl.kernel(
        body,
        out_type=(jax.ShapeDtypeStruct((_NC, _ND1P, _DW), jnp.float32),
                  jax.ShapeDtypeStruct((_NC, _ND2P, _DW), jnp.float32)),
        mesh=mesh,
        scratch_types=[
            pltpu.VMEM((_NCH1, _CH), jnp.int32),
            pltpu.VMEM((_NCH2, _CH), jnp.int32),
            pltpu.VMEM((_CH, _DW), jnp.float32),
            pltpu.VMEM((_ZR, _DW), jnp.float32),
            pltpu.VMEM_SHARED((_ND1P, _DW), jnp.float32),
            pltpu.VMEM_SHARED((_ND2P, _DW), jnp.float32),
            pltpu.SemaphoreType.DMA,
        ],
        compiler_params=pltpu.CompilerParams(use_tc_tiling_on_sc=False),
    )


def _sc_edge_agg(n_dst_pad, d, nch, tiled):
    rows_per_sub = n_dst_pad // _NS
    mesh = plsc.VectorSubcoreMesh(core_axis_name="c", subcore_axis_name="s")

    def body(m_hbm, src_hbm, dst_hbm, zeros_hbm, out_hbm, srcidx, dstidx,
             msgs0, msgs1, msgs2, msgs3, zbuf, agg_sh,
             gs0, gs1, gs2, gs3, ss0, ss1, ss2, ss3):
        msgs = [msgs0, msgs1, msgs2, msgs3]
        gs = [gs0, gs1, gs2, gs3]
        ss = [ss0, ss1, ss2, ss3]
        cid = lax.axis_index("c")
        sid = lax.axis_index("s")
        wid = sid * _NC + cid

        pltpu.sync_copy(zeros_hbm, zbuf)

        @pl.loop(0, rows_per_sub, step=_ZR)
        def _(r):
            pltpu.sync_copy(zbuf, agg_sh.at[pl.ds(sid * rows_per_sub + r, _ZR)])

        pltpu.sync_copy(src_hbm.at[wid], srcidx)
        pltpu.sync_copy(dst_hbm.at[wid], dstidx)
        plsc.subcore_barrier()

        for j in range(4):
            pltpu.async_copy(m_hbm.at[srcidx.at[j]], msgs[j], gs[j])

        @pl.loop(0, nch, step=4)
        def _(c):
            for j in range(4):
                jp = (j + 2) % 4
                pltpu.make_async_copy(m_hbm.at[srcidx.at[c + j]], msgs[j],
                                      gs[j]).wait()
                pltpu.async_copy(msgs[j], agg_sh.at[dstidx.at[c + j]], ss[j],
                                 add=True)

                @pl.when(c + j >= 2)
                def _():
                    pltpu.make_async_copy(
                        msgs[jp], agg_sh.at[dstidx.at[c + j - 2]],
                        ss[jp]).wait()

                @pl.when(jnp.logical_and(c + j >= 2, c + j + 2 < nch))
                def _():
                    pltpu.async_copy(m_hbm.at[srcidx.at[c + j + 2]], msgs[jp],
                                     gs[jp])

        pltpu.make_async_copy(msgs[2], agg_sh.at[dstidx.at[nch - 2]],
                              ss[2]).wait()
        pltpu.make_async_copy(msgs[3], agg_sh.at[dstidx.at[nch - 1]],
                              ss[3]).wait()
        plsc.subcore_barrier()
        pltpu.sync_copy(
            agg_sh.at[pl.ds(sid * rows_per_sub, rows_per_sub)],
            out_hbm.at[cid, pl.ds(sid * rows_per_sub, rows_per_sub)])

    return pl.kernel(
        body,
        out_type=jax.ShapeDtypeStruct((_NC, n_dst_pad, d), jnp.float32),
        mesh=mesh,
        scratch_types=[
            pltpu.VMEM((nch, _CH), jnp.int32),
            pltpu.VMEM((nch, _CH), jnp.int32),
            pltpu.VMEM((_CH, d), jnp.float32),
            pltpu.VMEM((_CH, d), jnp.float32),
            pltpu.VMEM((_CH, d), jnp.float32),
            pltpu.VMEM((_CH, d), jnp.float32),
            pltpu.VMEM((_ZR, d), jnp.float32),
            pltpu.VMEM_SHARED((n_dst_pad, d), jnp.float32),
            pltpu.SemaphoreType.DMA,
            pltpu.SemaphoreType.DMA,
            pltpu.SemaphoreType.DMA,
            pltpu.SemaphoreType.DMA,
            pltpu.SemaphoreType.DMA,
            pltpu.SemaphoreType.DMA,
            pltpu.SemaphoreType.DMA,
            pltpu.SemaphoreType.DMA,
        ],
        compiler_params=pltpu.CompilerParams(use_tc_tiling_on_sc=tiled),
    )


def _pad_edges(src, dst, e_pad, n_src, n_dst, n_dst_pad, nch):
    pad = e_pad - src.shape[0]
    i = jnp.arange(pad, dtype=jnp.int32)
    psrc = i % n_src
    pdst = n_dst + (i % (n_dst_pad - n_dst))
    src_p = jnp.concatenate([src, psrc]).reshape(_NW, nch, _CH)
    dst_p = jnp.concatenate([dst, pdst]).reshape(_NW, nch, _CH)
    return src_p, dst_p


def _tc_mm1(x, w1):
    def body(x_ref, w_ref, o_ref):
        o_ref[...] = jnp.dot(x_ref[...], w_ref[...],
                             preferred_element_type=jnp.float32,
                             precision=lax.Precision.DEFAULT)

    return pl.pallas_call(
        body,
        grid=(10,),
        in_specs=[pl.BlockSpec((1000, _IN), lambda i: (i, 0)),
                  pl.BlockSpec((_IN, _HID), lambda i: (0, 0))],
        out_specs=pl.BlockSpec((1000, _HID), lambda i: (i, 0)),
        out_shape=jax.ShapeDtypeStruct((_N_SRC1, _HID), jnp.float32),
    )(x, w1)


def _tc_mid(parts, degp, w2, b1):
    def body(p_ref, g_ref, w_ref, b_ref, o_ref):
        agg = p_ref[0, :_N_DST1] + p_ref[1, :_N_DST1]
        deg = g_ref[0, :_N_DST1, :1] + g_ref[1, :_N_DST1, :1]
        deg = jnp.maximum(deg, 1.0)
        h = agg / deg + b_ref[...]
        h = jnp.maximum(h, 0.0)
        m2 = jnp.dot(h, w_ref[...], preferred_element_type=jnp.float32,
                     precision=lax.Precision.DEFAULT)
        o_ref[:, :_CLS] = m2
        o_ref[:, _CLS:] = jnp.zeros((_N_DST1, _D2 - _CLS), jnp.float32)

    return pl.pallas_call(
        body,
        out_shape=jax.ShapeDtypeStruct((_N_DST1, _D2), jnp.float32),
    )(parts, degp, w2, b1.reshape(1, _HID))


def _tc_fin(parts, degp, b2):
    def body(p_ref, g_ref, b_ref, o_ref):
        agg = p_ref[0, :_N_DST2, :_CLS] + p_ref[1, :_N_DST2, :_CLS]
        deg = g_ref[0, :_N_DST2, :1] + g_ref[1, :_N_DST2, :1]
        deg = jnp.maximum(deg, 1.0)
        logits = agg / deg + b_ref[...]
        m = jnp.max(logits, axis=-1, keepdims=True)
        s = logits - m
        lse = jnp.log(jnp.sum(jnp.exp(s), axis=-1, keepdims=True))
        o_ref[...] = s - lse

    return pl.pallas_call(
        body,
        out_shape=jax.ShapeDtypeStruct((_N_DST2, _CLS), jnp.float32),
    )(parts, degp, b2.reshape(1, _CLS))


def kernel(x, edge_index_1, edge_index_2, W1, b1, W2, b2):
    s1, d1 = _pad_edges(edge_index_1[0], edge_index_1[1], _E1P,
                        _N_SRC1, _N_DST1, _ND1P, _NCH1)
    s2, d2 = _pad_edges(edge_index_2[0], edge_index_2[1], _E2P,
                        _N_DST1, _N_DST2, _ND2P, _NCH2)
    z1 = jnp.zeros((_ZR, _D1), jnp.float32)
    z2 = jnp.zeros((_ZR, _D2), jnp.float32)

    deg1p, deg2p = _sc_degrees()(d1, d2)
    m1 = _tc_mm1(x, W1)
    parts1 = _sc_edge_agg(_ND1P, _D1, _NCH1, True)(m1, s1, d1, z1)
    m2 = _tc_mid(parts1, deg1p, W2, b1)
    parts2 = _sc_edge_agg(_ND2P, _D2, _NCH2, False)(m2, s2, d2, z2)
    return _tc_fin(parts2, deg2p, b2)


# speedup vs baseline: 1.0713x; 1.0021x over previous
import functools

import jax
import jax.numpy as jnp
from jax import lax
from jax.experimental import pallas as pl
from jax.experimental.pallas import tpu as pltpu
from jax.experimental.pallas import tpu_sc as plsc

_N_SRC1 = 10000
_N_DST1 = 4000
_N_DST2 = 1000
_IN = 256
_HID = 128
_CLS = 40

_NC = 2
_NS = 16
_NW = _NC * _NS
_CH = 128

_E1P = 131072
_NCH1 = _E1P // (_NW * _CH)
_ND1P = 4096
_D1 = _HID

_E2P = 32768
_NCH2 = _E2P // (_NW * _CH)
_ND2P = 1024
_D2 = 48

_ZR = 64
_DW = 16


def _sc_degrees():
    mesh = plsc.VectorSubcoreMesh(core_axis_name="c", subcore_axis_name="s")
    rps1 = _ND1P // _NS
    rps2 = _ND2P // _NS

    def body(dst1_hbm, dst2_hbm, out1_hbm, out2_hbm, idx1, idx2, ones_v,
             zbuf, deg1_sh, deg2_sh, sem):
        cid = lax.axis_index("c")
        sid = lax.axis_index("s")
        wid = sid * _NC + cid

        @pl.loop(0, _ZR)
        def _(r):
            zbuf[r, pl.ds(0, _DW)] = jnp.zeros((_DW,), jnp.float32)

        @pl.loop(0, _CH)
        def _(r):
            ones_v[r, pl.ds(0, _DW)] = jnp.ones((_DW,), jnp.float32)

        @pl.loop(0, rps1, step=_ZR)
        def _(r):
            pltpu.sync_copy(zbuf, deg1_sh.at[pl.ds(sid * rps1 + r, _ZR)])

        pltpu.sync_copy(zbuf, deg2_sh.at[pl.ds(sid * rps2, _ZR)])
        pltpu.sync_copy(dst1_hbm.at[wid], idx1)
        pltpu.sync_copy(dst2_hbm.at[wid], idx2)
        plsc.subcore_barrier()

        @pl.loop(0, _NCH1)
        def _(c):
            pltpu.async_copy(ones_v, deg1_sh.at[idx1.at[c]], sem, add=True)

        @pl.loop(0, _NCH2)
        def _(c):
            pltpu.async_copy(ones_v, deg2_sh.at[idx2.at[c]], sem, add=True)

        @pl.loop(0, _NCH1 + _NCH2)
        def _(c):
            pltpu.make_async_copy(ones_v, deg1_sh.at[idx1.at[0]], sem).wait()

        plsc.subcore_barrier()
        pltpu.sync_copy(deg1_sh.at[pl.ds(sid * rps1, rps1)],
                        out1_hbm.at[cid, pl.ds(sid * rps1, rps1)])
        pltpu.sync_copy(deg2_sh.at[pl.ds(sid * rps2, rps2)],
                        out2_hbm.at[cid, pl.ds(sid * rps2, rps2)])

    return pl.kernel(
        body,
        out_type=(jax.ShapeDtypeStruct((_NC, _ND1P, _DW), jnp.float32),
                  jax.ShapeDtypeStruct((_NC, _ND2P, _DW), jnp.float32)),
        mesh=mesh,
        scratch_types=[
            pltpu.VMEM((_NCH1, _CH), jnp.int32),
            pltpu.VMEM((_NCH2, _CH), jnp.int32),
            pltpu.VMEM((_CH, _DW), jnp.float32),
            pltpu.VMEM((_ZR, _DW), jnp.float32),
            pltpu.VMEM_SHARED((_ND1P, _DW), jnp.float32),
            pltpu.VMEM_SHARED((_ND2P, _DW), jnp.float32),
            pltpu.SemaphoreType.DMA,
        ],
        compiler_params=pltpu.CompilerParams(use_tc_tiling_on_sc=False),
    )


def _sc_edge_agg(n_dst_pad, d, nch, tiled):
    rows_per_sub = n_dst_pad // _NS
    mesh = plsc.VectorSubcoreMesh(core_axis_name="c", subcore_axis_name="s")

    def body(m_hbm, src_hbm, dst_hbm, zeros_hbm, out_hbm, srcidx, dstidx,
             msgs0, msgs1, msgs2, msgs3, zbuf, agg_sh,
             gs0, gs1, gs2, gs3, ss0, ss1, ss2, ss3):
        msgs = [msgs0, msgs1, msgs2, msgs3]
        gs = [gs0, gs1, gs2, gs3]
        ss = [ss0, ss1, ss2, ss3]
        cid = lax.axis_index("c")
        sid = lax.axis_index("s")
        wid = sid * _NC + cid

        pltpu.sync_copy(src_hbm.at[wid], srcidx)
        pltpu.sync_copy(dst_hbm.at[wid], dstidx)
        for j in range(4):
            pltpu.async_copy(m_hbm.at[srcidx.at[j]], msgs[j], gs[j])

        pltpu.sync_copy(zeros_hbm, zbuf)

        @pl.loop(0, rows_per_sub, step=_ZR)
        def _(r):
            pltpu.async_copy(zbuf, agg_sh.at[pl.ds(sid * rows_per_sub + r, _ZR)],
                             ss[0])

        @pl.loop(0, rows_per_sub, step=_ZR)
        def _(r):
            pltpu.make_async_copy(zbuf, agg_sh.at[pl.ds(0, _ZR)], ss[0]).wait()

        plsc.subcore_barrier()

        @pl.loop(0, nch, step=4)
        def _(c):
            for j in range(4):
                jp = (j + 2) % 4
                pltpu.make_async_copy(m_hbm.at[srcidx.at[c + j]], msgs[j],
                                      gs[j]).wait()
                pltpu.async_copy(msgs[j], agg_sh.at[dstidx.at[c + j]], ss[j],
                                 add=True)

                @pl.when(c + j >= 2)
                def _():
                    pltpu.make_async_copy(
                        msgs[jp], agg_sh.at[dstidx.at[c + j - 2]],
                        ss[jp]).wait()

                @pl.when(jnp.logical_and(c + j >= 2, c + j + 2 < nch))
                def _():
                    pltpu.async_copy(m_hbm.at[srcidx.at[c + j + 2]], msgs[jp],
                                     gs[jp])

        pltpu.make_async_copy(msgs[2], agg_sh.at[dstidx.at[nch - 2]],
                              ss[2]).wait()
        pltpu.make_async_copy(msgs[3], agg_sh.at[dstidx.at[nch - 1]],
                              ss[3]).wait()
        plsc.subcore_barrier()
        pltpu.sync_copy(
            agg_sh.at[pl.ds(sid * rows_per_sub, rows_per_sub)],
            out_hbm.at[cid, pl.ds(sid * rows_per_sub, rows_per_sub)])

    return pl.kernel(
        body,
        out_type=jax.ShapeDtypeStruct((_NC, n_dst_pad, d), jnp.float32),
        mesh=mesh,
        scratch_types=[
            pltpu.VMEM((nch, _CH), jnp.int32),
            pltpu.VMEM((nch, _CH), jnp.int32),
            pltpu.VMEM((_CH, d), jnp.float32),
            pltpu.VMEM((_CH, d), jnp.float32),
            pltpu.VMEM((_CH, d), jnp.float32),
            pltpu.VMEM((_CH, d), jnp.float32),
            pltpu.VMEM((_ZR, d), jnp.float32),
            pltpu.VMEM_SHARED((n_dst_pad, d), jnp.float32),
            pltpu.SemaphoreType.DMA,
            pltpu.SemaphoreType.DMA,
            pltpu.SemaphoreType.DMA,
            pltpu.SemaphoreType.DMA,
            pltpu.SemaphoreType.DMA,
            pltpu.SemaphoreType.DMA,
            pltpu.SemaphoreType.DMA,
            pltpu.SemaphoreType.DMA,
        ],
        compiler_params=pltpu.CompilerParams(use_tc_tiling_on_sc=tiled),
    )


def _pad_edges(src, dst, e_pad, n_src, n_dst, n_dst_pad, nch):
    pad = e_pad - src.shape[0]
    i = jnp.arange(pad, dtype=jnp.int32)
    psrc = i % n_src
    pdst = n_dst + (i % (n_dst_pad - n_dst))
    src_p = jnp.concatenate([src, psrc]).reshape(_NW, nch, _CH)
    dst_p = jnp.concatenate([dst, pdst]).reshape(_NW, nch, _CH)
    return src_p, dst_p


def _tc_mm1(x, w1):
    def body(x_ref, w_ref, o_ref):
        o_ref[...] = jnp.dot(x_ref[...], w_ref[...],
                             preferred_element_type=jnp.float32,
                             precision=lax.Precision.DEFAULT)

    return pl.pallas_call(
        body,
        grid=(10,),
        in_specs=[pl.BlockSpec((1000, _IN), lambda i: (i, 0)),
                  pl.BlockSpec((_IN, _HID), lambda i: (0, 0))],
        out_specs=pl.BlockSpec((1000, _HID), lambda i: (i, 0)),
        out_shape=jax.ShapeDtypeStruct((_N_SRC1, _HID), jnp.float32),
    )(x, w1)


def _tc_mid(parts, degp, w2, b1):
    def body(p_ref, g_ref, w_ref, b_ref, o_ref):
        agg = p_ref[0, :_N_DST1] + p_ref[1, :_N_DST1]
        deg = g_ref[0, :_N_DST1, :1] + g_ref[1, :_N_DST1, :1]
        deg = jnp.maximum(deg, 1.0)
        h = agg / deg + b_ref[...]
        h = jnp.maximum(h, 0.0)
        m2 = jnp.dot(h, w_ref[...], preferred_element_type=jnp.float32,
                     precision=lax.Precision.DEFAULT)
        o_ref[:, :_CLS] = m2
        o_ref[:, _CLS:] = jnp.zeros((_N_DST1, _D2 - _CLS), jnp.float32)

    return pl.pallas_call(
        body,
        out_shape=jax.ShapeDtypeStruct((_N_DST1, _D2), jnp.float32),
    )(parts, degp, w2, b1.reshape(1, _HID))


def _tc_fin(parts, degp, b2):
    def body(p_ref, g_ref, b_ref, o_ref):
        agg = p_ref[0, :_N_DST2, :_CLS] + p_ref[1, :_N_DST2, :_CLS]
        deg = g_ref[0, :_N_DST2, :1] + g_ref[1, :_N_DST2, :1]
        deg = jnp.maximum(deg, 1.0)
        logits = agg / deg + b_ref[...]
        m = jnp.max(logits, axis=-1, keepdims=True)
        s = logits - m
        lse = jnp.log(jnp.sum(jnp.exp(s), axis=-1, keepdims=True))
        o_ref[...] = s - lse

    return pl.pallas_call(
        body,
        out_shape=jax.ShapeDtypeStruct((_N_DST2, _CLS), jnp.float32),
    )(parts, degp, b2.reshape(1, _CLS))


def kernel(x, edge_index_1, edge_index_2, W1, b1, W2, b2):
    s1, d1 = _pad_edges(edge_index_1[0], edge_index_1[1], _E1P,
                        _N_SRC1, _N_DST1, _ND1P, _NCH1)
    s2, d2 = _pad_edges(edge_index_2[0], edge_index_2[1], _E2P,
                        _N_DST1, _N_DST2, _ND2P, _NCH2)
    z1 = jnp.zeros((_ZR, _D1), jnp.float32)
    z2 = jnp.zeros((_ZR, _D2), jnp.float32)

    deg1p, deg2p = _sc_degrees()(d1, d2)
    m1 = _tc_mm1(x, W1)
    parts1 = _sc_edge_agg(_ND1P, _D1, _NCH1, True)(m1, s1, d1, z1)
    m2 = _tc_mid(parts1, deg1p, W2, b1)
    parts2 = _sc_edge_agg(_ND2P, _D2, _NCH2, False)(m2, s2, d2, z2)
    return _tc_fin(parts2, deg2p, b2)


# speedup vs baseline: 1.0784x; 1.0066x over previous
import functools

import jax
import jax.numpy as jnp
from jax import lax
from jax.experimental import pallas as pl
from jax.experimental.pallas import tpu as pltpu
from jax.experimental.pallas import tpu_sc as plsc

_N_SRC1 = 10000
_N_DST1 = 4000
_N_DST2 = 1000
_IN = 256
_HID = 128
_CLS = 40

_NC = 2
_NS = 16
_NW = _NC * _NS
_CH = 128

_E1P = 131072
_NCH1 = _E1P // (_NW * _CH)
_ND1P = 4096
_D1 = _HID

_E2P = 32768
_NCH2 = _E2P // (_NW * _CH)
_ND2P = 1024
_D2 = 48

_ZR = 64
_DW = 16


def _sc_degrees():
    mesh = plsc.VectorSubcoreMesh(core_axis_name="c", subcore_axis_name="s")
    rps1 = _ND1P // _NS
    rps2 = _ND2P // _NS

    def body(dst1_hbm, dst2_hbm, out1_hbm, out2_hbm, idx1, idx2, ones_v,
             zbuf, deg1_sh, deg2_sh, sem):
        cid = lax.axis_index("c")
        sid = lax.axis_index("s")
        wid = sid * _NC + cid

        @pl.loop(0, _ZR)
        def _(r):
            zbuf[r, pl.ds(0, _DW)] = jnp.zeros((_DW,), jnp.float32)

        @pl.loop(0, _CH)
        def _(r):
            ones_v[r, pl.ds(0, _DW)] = jnp.ones((_DW,), jnp.float32)

        @pl.loop(0, rps1, step=_ZR)
        def _(r):
            pltpu.sync_copy(zbuf, deg1_sh.at[pl.ds(sid * rps1 + r, _ZR)])

        pltpu.sync_copy(zbuf, deg2_sh.at[pl.ds(sid * rps2, _ZR)])
        pltpu.sync_copy(dst1_hbm.at[wid], idx1)
        pltpu.sync_copy(dst2_hbm.at[wid], idx2)
        plsc.subcore_barrier()

        @pl.loop(0, _NCH1)
        def _(c):
            pltpu.async_copy(ones_v, deg1_sh.at[idx1.at[c]], sem, add=True)

        @pl.loop(0, _NCH2)
        def _(c):
            pltpu.async_copy(ones_v, deg2_sh.at[idx2.at[c]], sem, add=True)

        @pl.loop(0, _NCH1 + _NCH2)
        def _(c):
            pltpu.make_async_copy(ones_v, deg1_sh.at[idx1.at[0]], sem).wait()

        plsc.subcore_barrier()
        pltpu.sync_copy(deg1_sh.at[pl.ds(sid * rps1, rps1)],
                        out1_hbm.at[cid, pl.ds(sid * rps1, rps1)])
        pltpu.sync_copy(deg2_sh.at[pl.ds(sid * rps2, rps2)],
                        out2_hbm.at[cid, pl.ds(sid * rps2, rps2)])

    return pl.kernel(
        body,
        out_type=(jax.ShapeDtypeStruct((_NC, _ND1P, _DW), jnp.float32),
                  jax.ShapeDtypeStruct((_NC, _ND2P, _DW), jnp.float32)),
        mesh=mesh,
        scratch_types=[
            pltpu.VMEM((_NCH1, _CH), jnp.int32),
            pltpu.VMEM((_NCH2, _CH), jnp.int32),
            pltpu.VMEM((_CH, _DW), jnp.float32),
            pltpu.VMEM((_ZR, _DW), jnp.float32),
            pltpu.VMEM_SHARED((_ND1P, _DW), jnp.float32),
            pltpu.VMEM_SHARED((_ND2P, _DW), jnp.float32),
            pltpu.SemaphoreType.DMA,
        ],
        compiler_params=pltpu.CompilerParams(use_tc_tiling_on_sc=False),
    )


def _sc_edge_agg(n_dst_pad, d, nch, tiled):
    rows_per_sub = n_dst_pad // _NS
    mesh = plsc.VectorSubcoreMesh(core_axis_name="c", subcore_axis_name="s")

    def body(m_hbm, src_hbm, dst_hbm, zeros_hbm, out_hbm, srcidx, dstidx,
             msgs0, msgs1, msgs2, msgs3, zbuf, agg_sh,
             gs0, gs1, gs2, gs3, ss0, ss1, ss2, ss3):
        msgs = [msgs0, msgs1, msgs2, msgs3]
        gs = [gs0, gs1, gs2, gs3]
        ss = [ss0, ss1, ss2, ss3]
        cid = lax.axis_index("c")
        sid = lax.axis_index("s")
        wid = sid * _NC + cid

        pltpu.sync_copy(src_hbm.at[wid], srcidx)
        pltpu.sync_copy(dst_hbm.at[wid], dstidx)
        for j in range(4):
            pltpu.async_copy(m_hbm.at[srcidx.at[j]], msgs[j], gs[j])

        pltpu.sync_copy(zeros_hbm, zbuf)

        @pl.loop(0, rows_per_sub, step=_ZR)
        def _(r):
            pltpu.async_copy(zbuf, agg_sh.at[pl.ds(sid * rows_per_sub + r, _ZR)],
                             ss[0])

        @pl.loop(0, rows_per_sub, step=_ZR)
        def _(r):
            pltpu.make_async_copy(zbuf, agg_sh.at[pl.ds(0, _ZR)], ss[0]).wait()

        plsc.subcore_barrier()

        @pl.loop(0, nch, step=4)
        def _(c):
            for j in range(4):
                jp = (j + 2) % 4
                pltpu.make_async_copy(m_hbm.at[srcidx.at[c + j]], msgs[j],
                                      gs[j]).wait()
                pltpu.async_copy(msgs[j], agg_sh.at[dstidx.at[c + j]], ss[j],
                                 add=True)

                @pl.when(c + j >= 2)
                def _():
                    pltpu.make_async_copy(
                        msgs[jp], agg_sh.at[dstidx.at[c + j - 2]],
                        ss[jp]).wait()

                @pl.when(jnp.logical_and(c + j >= 2, c + j + 2 < nch))
                def _():
                    pltpu.async_copy(m_hbm.at[srcidx.at[c + j + 2]], msgs[jp],
                                     gs[jp])

        pltpu.make_async_copy(msgs[2], agg_sh.at[dstidx.at[nch - 2]],
                              ss[2]).wait()
        pltpu.make_async_copy(msgs[3], agg_sh.at[dstidx.at[nch - 1]],
                              ss[3]).wait()
        plsc.subcore_barrier()
        pltpu.sync_copy(
            agg_sh.at[pl.ds(sid * rows_per_sub, rows_per_sub)],
            out_hbm.at[cid, pl.ds(sid * rows_per_sub, rows_per_sub)])

    return pl.kernel(
        body,
        out_type=jax.ShapeDtypeStruct((_NC, n_dst_pad, d), jnp.float32),
        mesh=mesh,
        scratch_types=[
            pltpu.VMEM((nch, _CH), jnp.int32),
            pltpu.VMEM((nch, _CH), jnp.int32),
            pltpu.VMEM((_CH, d), jnp.float32),
            pltpu.VMEM((_CH, d), jnp.float32),
            pltpu.VMEM((_CH, d), jnp.float32),
            pltpu.VMEM((_CH, d), jnp.float32),
            pltpu.VMEM((_ZR, d), jnp.float32),
            pltpu.VMEM_SHARED((n_dst_pad, d), jnp.float32),
            pltpu.SemaphoreType.DMA,
            pltpu.SemaphoreType.DMA,
            pltpu.SemaphoreType.DMA,
            pltpu.SemaphoreType.DMA,
            pltpu.SemaphoreType.DMA,
            pltpu.SemaphoreType.DMA,
            pltpu.SemaphoreType.DMA,
            pltpu.SemaphoreType.DMA,
        ],
        compiler_params=pltpu.CompilerParams(use_tc_tiling_on_sc=tiled),
    )


def _pad_edges(src, dst, e_pad, n_src, n_dst, n_dst_pad, nch):
    pad = e_pad - src.shape[0]
    i = jnp.arange(pad, dtype=jnp.int32)
    psrc = i % n_src
    pdst = n_dst + (i % (n_dst_pad - n_dst))
    src_p = jnp.concatenate([src, psrc]).reshape(_NW, nch, _CH)
    dst_p = jnp.concatenate([dst, pdst]).reshape(_NW, nch, _CH)
    return src_p, dst_p


def _tc_mm1(x, w1):
    def body(x_ref, w_ref, o_ref):
        o_ref[...] = jnp.dot(x_ref[...], w_ref[...],
                             preferred_element_type=jnp.float32,
                             precision=lax.Precision.DEFAULT)

    return pl.pallas_call(
        body,
        grid=(5,),
        in_specs=[pl.BlockSpec((2000, _IN), lambda i: (i, 0)),
                  pl.BlockSpec((_IN, _HID), lambda i: (0, 0))],
        out_specs=pl.BlockSpec((2000, _HID), lambda i: (i, 0)),
        out_shape=jax.ShapeDtypeStruct((_N_SRC1, _HID), jnp.float32),
    )(x, w1)


def _tc_mid(parts, degp, w2, b1):
    def body(p_ref, g_ref, w_ref, b_ref, o_ref):
        agg = p_ref[0, :_N_DST1] + p_ref[1, :_N_DST1]
        deg = g_ref[0, :_N_DST1, :1] + g_ref[1, :_N_DST1, :1]
        deg = jnp.maximum(deg, 1.0)
        h = agg / deg + b_ref[...]
        h = jnp.maximum(h, 0.0)
        m2 = jnp.dot(h, w_ref[...], preferred_element_type=jnp.float32,
                     precision=lax.Precision.DEFAULT)
        o_ref[:, :_CLS] = m2
        o_ref[:, _CLS:] = jnp.zeros((_N_DST1, _D2 - _CLS), jnp.float32)

    return pl.pallas_call(
        body,
        out_shape=jax.ShapeDtypeStruct((_N_DST1, _D2), jnp.float32),
    )(parts, degp, w2, b1.reshape(1, _HID))


def _tc_fin(parts, degp, b2):
    def body(p_ref, g_ref, b_ref, o_ref):
        agg = p_ref[0, :_N_DST2, :_CLS] + p_ref[1, :_N_DST2, :_CLS]
        deg = g_ref[0, :_N_DST2, :1] + g_ref[1, :_N_DST2, :1]
        deg = jnp.maximum(deg, 1.0)
        logits = agg / deg + b_ref[...]
        m = jnp.max(logits, axis=-1, keepdims=True)
        s = logits - m
        lse = jnp.log(jnp.sum(jnp.exp(s), axis=-1, keepdims=True))
        o_ref[...] = s - lse

    return pl.pallas_call(
        body,
        out_shape=jax.ShapeDtypeStruct((_N_DST2, _CLS), jnp.float32),
    )(parts, degp, b2.reshape(1, _CLS))


def kernel(x, edge_index_1, edge_index_2, W1, b1, W2, b2):
    s1, d1 = _pad_edges(edge_index_1[0], edge_index_1[1], _E1P,
                        _N_SRC1, _N_DST1, _ND1P, _NCH1)
    s2, d2 = _pad_edges(edge_index_2[0], edge_index_2[1], _E2P,
                        _N_DST1, _N_DST2, _ND2P, _NCH2)
    z1 = jnp.zeros((_ZR, _D1), jnp.float32)
    z2 = jnp.zeros((_ZR, _D2), jnp.float32)

    deg1p, deg2p = _sc_degrees()(d1, d2)
    m1 = _tc_mm1(x, W1)
    parts1 = _sc_edge_agg(_ND1P, _D1, _NCH1, True)(m1, s1, d1, z1)
    m2 = _tc_mid(parts1, deg1p, W2, b1)
    parts2 = _sc_edge_agg(_ND2P, _D2, _NCH2, False)(m2, s2, d2, z2)
    return _tc_fin(parts2, deg2p, b2)
